# baseline (speedup 1.0000x reference)
import jax
from jax.experimental import pallas as pl
from jax.experimental.pallas import tpu as pltpu

_GRID = 50
_X_ROWS = 10000 // _GRID
_E_ROWS = 320000 // _GRID


def _copy_body(x_ref, e_ref, u_ref, xo_ref, eo_ref, uo_ref):
    xo_ref[...] = x_ref[...]
    eo_ref[...] = e_ref[...]
    uo_ref[...] = u_ref[...]


def kernel(x, edge_index, edge_attr, u, batch):
    del edge_index, batch
    return pl.pallas_call(
        _copy_body,
        grid=(_GRID,),
        out_shape=(
            jax.ShapeDtypeStruct(x.shape, x.dtype),
            jax.ShapeDtypeStruct(edge_attr.shape, edge_attr.dtype),
            jax.ShapeDtypeStruct(u.shape, u.dtype),
        ),
        in_specs=[
            pl.BlockSpec((_X_ROWS, 128), lambda i: (i, 0)),
            pl.BlockSpec((_E_ROWS, 16), lambda i: (i, 0)),
            pl.BlockSpec((64, 64), lambda i: (0, 0)),
        ],
        out_specs=(
            pl.BlockSpec((_X_ROWS, 128), lambda i: (i, 0)),
            pl.BlockSpec((_E_ROWS, 16), lambda i: (i, 0)),
            pl.BlockSpec((64, 64), lambda i: (0, 0)),
        ),
        compiler_params=pltpu.CompilerParams(
            dimension_semantics=("parallel",),
        ),
    )(x, edge_attr, u)
